# Initial kernel scaffold; baseline (speedup 1.0000x reference)
import jax
import jax.numpy as jnp
from jax.experimental import pallas as pl


def kernel(hidden_states, top_k_index, top_k_weights, Wg, Wu, Wd):
    raise NotImplementedError("write your pallas kernel here")


# speedup vs baseline: 1.4203x; 1.4203x over previous
import functools

import jax
import jax.numpy as jnp
from jax import lax
from jax.experimental import pallas as pl
from jax.experimental.pallas import tpu as pltpu
from jax.experimental.pallas import tpu_sc as plsc

_BLOCK = 256
_TI = 512
_NW = 32
_CH = 32


def _sc_gather(src, idx):
    n_rows = idx.shape[0]
    h = src.shape[1]
    per_w = n_rows // _NW
    mesh = plsc.VectorSubcoreMesh(core_axis_name="c", subcore_axis_name="s")

    @functools.partial(
        pl.kernel,
        mesh=mesh,
        out_type=jax.ShapeDtypeStruct((n_rows, h), src.dtype),
        scratch_types=[
            pltpu.VMEM((_CH,), jnp.int32),
            pltpu.VMEM((_CH, h), src.dtype),
            pltpu.SemaphoreType.DMA,
        ],
    )
    def gather_k(src_hbm, idx_hbm, out_hbm, idx_v, rows_v, sem):
        wid = lax.axis_index("s") * 2 + lax.axis_index("c")
        base = wid * per_w

        def body(j, carry):
            off = base + j * _CH
            pltpu.sync_copy(idx_hbm.at[pl.ds(off, _CH)], idx_v)
            pltpu.async_copy(src_hbm.at[idx_v], rows_v, sem).wait()
            pltpu.sync_copy(rows_v, out_hbm.at[pl.ds(off, _CH)])
            return carry

        lax.fori_loop(0, per_w // _CH, body, 0)

    return gather_k(src, idx)


def _grouped_mlp(x_sorted, Wg, Wu, Wd, row_w3, block_expert):
    p, h = x_sorted.shape
    e_, i_, _ = Wg.shape
    nb = p // _BLOCK
    ni = i_ // _TI

    def body(be_ref, x_ref, wg_ref, wu_ref, wd_ref, w_ref, y_ref):
        i = pl.program_id(1)
        x = x_ref[...]
        gate = lax.dot_general(x, wg_ref[0], (((1,), (1,)), ((), ())),
                               preferred_element_type=jnp.float32)
        up = lax.dot_general(x, wu_ref[0], (((1,), (1,)), ((), ())),
                             preferred_element_type=jnp.float32)
        hmid = gate * jax.nn.sigmoid(gate) * up
        part = lax.dot_general(hmid, wd_ref[0], (((1,), (1,)), ((), ())),
                               preferred_element_type=jnp.float32)

        @pl.when(i == 0)
        def _():
            y_ref[...] = jnp.zeros_like(y_ref)

        y_ref[...] += part

        @pl.when(i == ni - 1)
        def _():
            y_ref[...] *= w_ref[0, 0, :][:, None]

    grid_spec = pltpu.PrefetchScalarGridSpec(
        num_scalar_prefetch=1,
        grid=(nb, ni),
        in_specs=[
            pl.BlockSpec((_BLOCK, h), lambda b, i, be: (b, 0)),
            pl.BlockSpec((1, _TI, h), lambda b, i, be: (be[b], i, 0)),
            pl.BlockSpec((1, _TI, h), lambda b, i, be: (be[b], i, 0)),
            pl.BlockSpec((1, h, _TI), lambda b, i, be: (be[b], 0, i)),
            pl.BlockSpec((1, 1, _BLOCK), lambda b, i, be: (b, 0, 0)),
        ],
        out_specs=pl.BlockSpec((_BLOCK, h), lambda b, i, be: (b, 0)),
    )
    return pl.pallas_call(
        body,
        grid_spec=grid_spec,
        out_shape=jax.ShapeDtypeStruct((p, h), jnp.float32),
        compiler_params=pltpu.CompilerParams(
            dimension_semantics=("parallel", "arbitrary")),
    )(block_expert, x_sorted, Wg, Wu, Wd, row_w3)


def _pair_sum(z2, h):
    t = z2.shape[0]
    bt = 256

    def body(z_ref, o_ref):
        o_ref[...] = z_ref[:, :h] + z_ref[:, h:]

    return pl.pallas_call(
        body,
        grid=(t // bt,),
        in_specs=[pl.BlockSpec((bt, 2 * h), lambda b: (b, 0))],
        out_specs=pl.BlockSpec((bt, h), lambda b: (b, 0)),
        out_shape=jax.ShapeDtypeStruct((t, h), jnp.float32),
    )(z2)


def _routing(top_k_index, top_k_weights, e_):
    t, k = top_k_index.shape
    tk = t * k
    nb = tk // _BLOCK + e_
    p = nb * _BLOCK

    flat_e = top_k_index.reshape(tk).astype(jnp.int32)
    flat_w = top_k_weights.reshape(tk)
    order = jnp.argsort(flat_e).astype(jnp.int32)
    sorted_e = flat_e[order]

    counts = jnp.zeros((e_,), jnp.int32).at[flat_e].add(1)
    raw_start = jnp.concatenate(
        [jnp.zeros((1,), jnp.int32), jnp.cumsum(counts)[:-1]])
    padded_counts = ((counts + _BLOCK - 1) // _BLOCK) * _BLOCK
    padded_end = jnp.cumsum(padded_counts)
    padded_start = padded_end - padded_counts

    dst = padded_start[sorted_e] + (
        jnp.arange(tk, dtype=jnp.int32) - raw_start[sorted_e])

    src_tok = jnp.zeros((p,), jnp.int32).at[dst].set(
        (order // k).astype(jnp.int32))
    row_w = jnp.zeros((p,), jnp.float32).at[dst].set(flat_w[order])
    block_expert = jnp.minimum(
        jnp.searchsorted(padded_end,
                         jnp.arange(nb, dtype=jnp.int32) * _BLOCK,
                         side="right"),
        e_ - 1).astype(jnp.int32)
    inv = jnp.zeros((tk,), jnp.int32).at[order].set(dst)
    return src_tok, row_w.reshape(nb, 1, _BLOCK), block_expert, inv


def kernel(hidden_states, top_k_index, top_k_weights, Wg, Wu, Wd):
    t, h = hidden_states.shape
    e_ = Wg.shape[0]
    src_tok, row_w3, block_expert, inv = _routing(
        top_k_index, top_k_weights, e_)
    x_sorted = _sc_gather(hidden_states, src_tok)
    y_sorted = _grouped_mlp(x_sorted, Wg, Wu, Wd, row_w3, block_expert)
    z = _sc_gather(y_sorted, inv)
    return _pair_sum(z.reshape(t, 2 * h), h)
